# initial kernel scaffold (unmeasured)
import jax
import jax.numpy as jnp
from jax import lax
from jax.experimental import pallas as pl
from jax.experimental.pallas import tpu as pltpu

N_DEV = 4


def kernel(x, w_mat, scale_x, scale_w):
    m_global, _ = x.shape
    _, n = w_mat.shape
    m_per = m_global // N_DEV

    def body(x_ref, w_ref, sx_ref, sw_ref, out_ref, comm_ref, send_sems, recv_sems):
        my = lax.axis_index("i")
        left = lax.rem(my + (N_DEV - 1), N_DEV)
        right = lax.rem(my + 1, N_DEV)

        barrier_sem = pltpu.get_barrier_semaphore()
        pl.semaphore_signal(barrier_sem, inc=1, device_id=(left,),
                            device_id_type=pl.DeviceIdType.MESH)
        pl.semaphore_signal(barrier_sem, inc=1, device_id=(right,),
                            device_id_type=pl.DeviceIdType.MESH)
        pl.semaphore_wait(barrier_sem, 2)

        w = w_ref[:, :].astype(jnp.bfloat16)

        def partial_chunk(c):
            xc = x_ref[pl.ds(c * m_per, m_per), :].astype(jnp.bfloat16)
            return jnp.dot(xc, w, preferred_element_type=jnp.float32)

        comm_ref[0, :, :] = partial_chunk(left).astype(jnp.bfloat16)

        for s in range(N_DEV - 1):
            rdma = pltpu.make_async_remote_copy(
                src_ref=comm_ref.at[s],
                dst_ref=comm_ref.at[s + 1],
                send_sem=send_sems.at[s],
                recv_sem=recv_sems.at[s],
                device_id=(right,),
                device_id_type=pl.DeviceIdType.MESH,
            )
            rdma.start()
            rdma.wait()
            d = lax.rem(my + (2 * N_DEV - 2 - s), N_DEV)
            acc = comm_ref[s + 1, :, :].astype(jnp.float32) + partial_chunk(d)
            if s < N_DEV - 2:
                comm_ref[s + 1, :, :] = acc.astype(jnp.bfloat16)
            else:
                out_ref[:, :] = acc * (sx_ref[0] * sw_ref[0])

    return pl.pallas_call(
        body,
        out_shape=jax.ShapeDtypeStruct((m_per, n), jnp.float32),
        in_specs=[
            pl.BlockSpec(memory_space=pltpu.VMEM),
            pl.BlockSpec(memory_space=pltpu.VMEM),
            pl.BlockSpec(memory_space=pltpu.SMEM),
            pl.BlockSpec(memory_space=pltpu.SMEM),
        ],
        out_specs=pl.BlockSpec(memory_space=pltpu.VMEM),
        scratch_shapes=[
            pltpu.VMEM((N_DEV, m_per, n), jnp.bfloat16),
            pltpu.SemaphoreType.DMA((N_DEV - 1,)),
            pltpu.SemaphoreType.DMA((N_DEV - 1,)),
        ],
        compiler_params=pltpu.CompilerParams(collective_id=0),
    )(x, w_mat, scale_x, scale_w)


# baseline (device time: 180422 ns/iter reference)
import jax
import jax.numpy as jnp
from jax import lax
from jax.experimental import pallas as pl
from jax.experimental.pallas import tpu as pltpu

N_DEV = 4


def kernel(x, w_mat, scale_x, scale_w):
    m_global, _ = x.shape
    _, n = w_mat.shape
    m_per = m_global // N_DEV

    def body(x_ref, w_ref, sx_ref, sw_ref, out_ref, comm_ref, send_sems, recv_sems):
        my = lax.axis_index("i")
        left = lax.rem(my + (N_DEV - 1), N_DEV)
        right = lax.rem(my + 1, N_DEV)

        barrier_sem = pltpu.get_barrier_semaphore()
        pl.semaphore_signal(barrier_sem, inc=1, device_id=(left,),
                            device_id_type=pl.DeviceIdType.MESH)
        pl.semaphore_signal(barrier_sem, inc=1, device_id=(right,),
                            device_id_type=pl.DeviceIdType.MESH)
        pl.semaphore_wait(barrier_sem, 2)

        w = w_ref[:, :].astype(jnp.bfloat16)

        def partial_chunk(c):
            xc = x_ref[pl.ds(c * m_per, m_per), :].astype(jnp.bfloat16)
            return jnp.dot(xc, w, preferred_element_type=jnp.float32)

        comm_ref[0, :, :] = partial_chunk(left).astype(jnp.bfloat16)

        for s in range(N_DEV - 1):
            rdma = pltpu.make_async_remote_copy(
                src_ref=comm_ref.at[s],
                dst_ref=comm_ref.at[s + 1],
                send_sem=send_sems.at[s],
                recv_sem=recv_sems.at[s],
                device_id=(right,),
                device_id_type=pl.DeviceIdType.MESH,
            )
            rdma.start()
            rdma.wait()
            d = lax.rem(my + (2 * N_DEV - 2 - s), N_DEV)
            acc = comm_ref[s + 1, :, :].astype(jnp.float32) + partial_chunk(d)
            if s < N_DEV - 2:
                comm_ref[s + 1, :, :] = acc.astype(jnp.bfloat16)
            else:
                out_ref[:, :] = acc * (sx_ref[0] * sw_ref[0])

    return pl.pallas_call(
        body,
        out_shape=jax.ShapeDtypeStruct((m_per, n), jnp.float32),
        in_specs=[
            pl.BlockSpec(memory_space=pltpu.VMEM),
            pl.BlockSpec(memory_space=pltpu.VMEM),
            pl.BlockSpec(memory_space=pltpu.SMEM),
            pl.BlockSpec(memory_space=pltpu.SMEM),
        ],
        out_specs=pl.BlockSpec(memory_space=pltpu.VMEM),
        scratch_shapes=[
            pltpu.VMEM((N_DEV, m_per, n), jnp.bfloat16),
            pltpu.SemaphoreType.DMA((N_DEV - 1,)),
            pltpu.SemaphoreType.DMA((N_DEV - 1,)),
        ],
        compiler_params=pltpu.CompilerParams(
            collective_id=0,
            vmem_limit_bytes=100 * 1024 * 1024,
        ),
    )(x, w_mat, scale_x, scale_w)


# device time: 113974 ns/iter; 1.5830x vs baseline; 1.5830x over previous
import jax
import jax.numpy as jnp
from jax import lax
from jax.experimental import pallas as pl
from jax.experimental.pallas import tpu as pltpu

N_DEV = 4
F8X = jnp.float8_e4m3fn
F8W = jnp.float8_e5m2


def kernel(x, w_mat, scale_x, scale_w):
    m_global, k_per = x.shape
    _, n = w_mat.shape
    m_per = m_global // N_DEV

    def body(x_ref, w_ref, sx_ref, sw_ref, out_ref,
             rx_x, rx_w, tx_x_sems, tx_w_sems, rx_x_sems, rx_w_sems):
        my = lax.axis_index("i")

        barrier_sem = pltpu.get_barrier_semaphore()
        for j in range(1, N_DEV):
            peer = lax.rem(my + j, N_DEV)
            pl.semaphore_signal(barrier_sem, inc=1, device_id=(peer,),
                                device_id_type=pl.DeviceIdType.MESH)
        pl.semaphore_wait(barrier_sem, N_DEV - 1)

        sends = []
        for j in range(1, N_DEV):
            peer = lax.rem(my + j, N_DEV)
            rdma_x = pltpu.make_async_remote_copy(
                src_ref=x_ref.at[pl.ds(peer * m_per, m_per)],
                dst_ref=rx_x.at[N_DEV - 1 - j],
                send_sem=tx_x_sems.at[j - 1],
                recv_sem=rx_x_sems.at[N_DEV - 1 - j],
                device_id=(peer,),
                device_id_type=pl.DeviceIdType.MESH,
            )
            rdma_x.start()
            rdma_w = pltpu.make_async_remote_copy(
                src_ref=w_ref,
                dst_ref=rx_w.at[N_DEV - 1 - j],
                send_sem=tx_w_sems.at[j - 1],
                recv_sem=rx_w_sems.at[N_DEV - 1 - j],
                device_id=(peer,),
                device_id_type=pl.DeviceIdType.MESH,
            )
            rdma_w.start()
            sends += [rdma_x, rdma_w]

        out_ref[:, :] = jnp.dot(
            x_ref[pl.ds(my * m_per, m_per), :].astype(jnp.bfloat16),
            w_ref[:, :].astype(jnp.bfloat16),
            preferred_element_type=jnp.float32,
        )

        for slot in (0, 2, 1):
            recv_x = pltpu.make_async_remote_copy(
                src_ref=rx_x.at[slot], dst_ref=rx_x.at[slot],
                send_sem=tx_x_sems.at[0], recv_sem=rx_x_sems.at[slot],
                device_id=(my,), device_id_type=pl.DeviceIdType.MESH,
            )
            recv_x.wait_recv()
            recv_w = pltpu.make_async_remote_copy(
                src_ref=rx_w.at[slot], dst_ref=rx_w.at[slot],
                send_sem=tx_w_sems.at[0], recv_sem=rx_w_sems.at[slot],
                device_id=(my,), device_id_type=pl.DeviceIdType.MESH,
            )
            recv_w.wait_recv()
            contrib = jnp.dot(
                rx_x[slot, :, :].astype(jnp.bfloat16),
                rx_w[slot, :, :].astype(jnp.bfloat16),
                preferred_element_type=jnp.float32,
            )
            if slot == 1:
                out_ref[:, :] = (out_ref[:, :] + contrib) * (sx_ref[0] * sw_ref[0])
            else:
                out_ref[:, :] = out_ref[:, :] + contrib

        for s in sends:
            s.wait_send()

    f8_kernel = pl.pallas_call(
        body,
        out_shape=jax.ShapeDtypeStruct((m_per, n), jnp.float32),
        in_specs=[
            pl.BlockSpec(memory_space=pltpu.VMEM),
            pl.BlockSpec(memory_space=pltpu.VMEM),
            pl.BlockSpec(memory_space=pltpu.SMEM),
            pl.BlockSpec(memory_space=pltpu.SMEM),
        ],
        out_specs=pl.BlockSpec(memory_space=pltpu.VMEM),
        scratch_shapes=[
            pltpu.VMEM((N_DEV - 1, m_per, k_per), F8X),
            pltpu.VMEM((N_DEV - 1, k_per, n), F8W),
            pltpu.SemaphoreType.DMA((N_DEV - 1,)),
            pltpu.SemaphoreType.DMA((N_DEV - 1,)),
            pltpu.SemaphoreType.DMA((N_DEV - 1,)),
            pltpu.SemaphoreType.DMA((N_DEV - 1,)),
        ],
        compiler_params=pltpu.CompilerParams(
            collective_id=0,
            vmem_limit_bytes=60 * 1024 * 1024,
        ),
    )

    return f8_kernel(x.astype(F8X), w_mat.astype(F8W), scale_x, scale_w)


# device time: 104925 ns/iter; 1.7195x vs baseline; 1.0862x over previous
import jax
import jax.numpy as jnp
from jax import lax
from jax.experimental import pallas as pl
from jax.experimental.pallas import tpu as pltpu

N_DEV = 4
F8X = jnp.float8_e4m3fn
F8W = jnp.float8_e5m2


def kernel(x, w_mat, scale_x, scale_w):
    m_global, k_per = x.shape
    _, n = w_mat.shape
    m_per = m_global // N_DEV

    def body(x_ref, w_ref, sx_ref, sw_ref, out_ref,
             tx_x, tx_w, rx_x, rx_w,
             tx_x_sems, tx_w_sems, rx_x_sems, rx_w_sems):
        my = lax.axis_index("i")

        barrier_sem = pltpu.get_barrier_semaphore()
        for j in range(1, N_DEV):
            peer = lax.rem(my + j, N_DEV)
            pl.semaphore_signal(barrier_sem, inc=1, device_id=(peer,),
                                device_id_type=pl.DeviceIdType.MESH)
        pl.semaphore_wait(barrier_sem, N_DEV - 1)

        sends = []

        def send(src, dst, send_sem, recv_sem, peer):
            rdma = pltpu.make_async_remote_copy(
                src_ref=src, dst_ref=dst, send_sem=send_sem,
                recv_sem=recv_sem, device_id=(peer,),
                device_id_type=pl.DeviceIdType.MESH,
            )
            rdma.start()
            sends.append(rdma)

        for j in range(1, N_DEV):
            peer = lax.rem(my + j, N_DEV)
            tx_x[j - 1, :, :] = x_ref[pl.ds(peer * m_per, m_per), :].astype(F8X)
            send(tx_x.at[j - 1], rx_x.at[N_DEV - 1 - j],
                 tx_x_sems.at[j - 1], rx_x_sems.at[N_DEV - 1 - j], peer)
        tx_w[:, :] = w_ref[:, :].astype(F8W)
        for j in range(1, N_DEV):
            peer = lax.rem(my + j, N_DEV)
            send(tx_w, rx_w.at[N_DEV - 1 - j],
                 tx_w_sems.at[j - 1], rx_w_sems.at[N_DEV - 1 - j], peer)

        out_ref[:, :] = jnp.dot(
            x_ref[pl.ds(my * m_per, m_per), :].astype(jnp.bfloat16),
            w_ref[:, :].astype(jnp.bfloat16),
            preferred_element_type=jnp.float32,
        )

        for slot in (0, 2, 1):
            recv_x = pltpu.make_async_remote_copy(
                src_ref=rx_x.at[slot], dst_ref=rx_x.at[slot],
                send_sem=tx_x_sems.at[0], recv_sem=rx_x_sems.at[slot],
                device_id=(my,), device_id_type=pl.DeviceIdType.MESH,
            )
            recv_x.wait_recv()
            recv_w = pltpu.make_async_remote_copy(
                src_ref=rx_w.at[slot], dst_ref=rx_w.at[slot],
                send_sem=tx_w_sems.at[0], recv_sem=rx_w_sems.at[slot],
                device_id=(my,), device_id_type=pl.DeviceIdType.MESH,
            )
            recv_w.wait_recv()
            contrib = jnp.dot(
                rx_x[slot, :, :].astype(jnp.bfloat16),
                rx_w[slot, :, :].astype(jnp.bfloat16),
                preferred_element_type=jnp.float32,
            )
            if slot == 1:
                out_ref[:, :] = (out_ref[:, :] + contrib) * (sx_ref[0] * sw_ref[0])
            else:
                out_ref[:, :] = out_ref[:, :] + contrib

        for s in sends:
            s.wait_send()

    return pl.pallas_call(
        body,
        out_shape=jax.ShapeDtypeStruct((m_per, n), jnp.float32),
        in_specs=[
            pl.BlockSpec(memory_space=pltpu.VMEM),
            pl.BlockSpec(memory_space=pltpu.VMEM),
            pl.BlockSpec(memory_space=pltpu.SMEM),
            pl.BlockSpec(memory_space=pltpu.SMEM),
        ],
        out_specs=pl.BlockSpec(memory_space=pltpu.VMEM),
        scratch_shapes=[
            pltpu.VMEM((N_DEV - 1, m_per, k_per), F8X),
            pltpu.VMEM((k_per, n), F8W),
            pltpu.VMEM((N_DEV - 1, m_per, k_per), F8X),
            pltpu.VMEM((N_DEV - 1, k_per, n), F8W),
            pltpu.SemaphoreType.DMA((N_DEV - 1,)),
            pltpu.SemaphoreType.DMA((N_DEV - 1,)),
            pltpu.SemaphoreType.DMA((N_DEV - 1,)),
            pltpu.SemaphoreType.DMA((N_DEV - 1,)),
        ],
        compiler_params=pltpu.CompilerParams(
            collective_id=0,
            vmem_limit_bytes=62 * 1024 * 1024,
        ),
    )(x, w_mat, scale_x, scale_w)


# device time: 103405 ns/iter; 1.7448x vs baseline; 1.0147x over previous
import jax
import jax.numpy as jnp
from jax import lax
from jax.experimental import pallas as pl
from jax.experimental.pallas import tpu as pltpu

N_DEV = 4
F8X = jnp.float8_e4m3fn
F8W = jnp.float8_e5m2


def kernel(x, w_mat, scale_x, scale_w):
    m_global, k_per = x.shape
    _, n = w_mat.shape
    m_per = m_global // N_DEV
    n2 = n // 2

    def body(x_ref, w_ref, sx_ref, sw_ref, out_ref,
             tx_x, tx_w, rx_x, rx_w,
             tx_x_sems, tx_w_sems, rx_x_sems, rx_w_sems):
        my = lax.axis_index("i")

        barrier_sem = pltpu.get_barrier_semaphore()
        for j in range(1, N_DEV):
            peer = lax.rem(my + j, N_DEV)
            pl.semaphore_signal(barrier_sem, inc=1, device_id=(peer,),
                                device_id_type=pl.DeviceIdType.MESH)
        pl.semaphore_wait(barrier_sem, N_DEV - 1)

        sends = []

        def send(src, dst, send_sem, recv_sem, peer):
            rdma = pltpu.make_async_remote_copy(
                src_ref=src, dst_ref=dst, send_sem=send_sem,
                recv_sem=recv_sem, device_id=(peer,),
                device_id_type=pl.DeviceIdType.MESH,
            )
            rdma.start()
            sends.append(rdma)

        tx_w[:, :] = w_ref[:, :].astype(F8W)
        for j in range(1, N_DEV):
            peer = lax.rem(my + j, N_DEV)
            send(tx_w.at[:, pl.ds(0, n2)],
                 rx_w.at[N_DEV - 1 - j, :, pl.ds(0, n2)],
                 tx_w_sems.at[j - 1, 0], rx_w_sems.at[N_DEV - 1 - j, 0], peer)
        for j in range(1, N_DEV):
            peer = lax.rem(my + j, N_DEV)
            tx_x[j - 1, :, :] = x_ref[pl.ds(peer * m_per, m_per), :].astype(F8X)
            send(tx_x.at[j - 1], rx_x.at[N_DEV - 1 - j],
                 tx_x_sems.at[j - 1], rx_x_sems.at[N_DEV - 1 - j], peer)
        for j in range(1, N_DEV):
            peer = lax.rem(my + j, N_DEV)
            send(tx_w.at[:, pl.ds(n2, n2)],
                 rx_w.at[N_DEV - 1 - j, :, pl.ds(n2, n2)],
                 tx_w_sems.at[j - 1, 1], rx_w_sems.at[N_DEV - 1 - j, 1], peer)

        out_ref[:, :] = jnp.dot(
            x_ref[pl.ds(my * m_per, m_per), :].astype(jnp.bfloat16),
            w_ref[:, :].astype(jnp.bfloat16),
            preferred_element_type=jnp.float32,
        )

        def wait_recv(dst, sem):
            rdma = pltpu.make_async_remote_copy(
                src_ref=dst, dst_ref=dst, send_sem=tx_x_sems.at[0],
                recv_sem=sem, device_id=(my,),
                device_id_type=pl.DeviceIdType.MESH,
            )
            rdma.wait_recv()

        for slot in (0, 2, 1):
            wait_recv(rx_x.at[slot], rx_x_sems.at[slot])
            xb = rx_x[slot, :, :].astype(jnp.bfloat16)
            for h in (0, 1):
                wait_recv(rx_w.at[slot, :, pl.ds(h * n2, n2)],
                          rx_w_sems.at[slot, h])
                contrib = jnp.dot(
                    xb,
                    rx_w[slot, :, pl.ds(h * n2, n2)].astype(jnp.bfloat16),
                    preferred_element_type=jnp.float32,
                )
                cols = pl.ds(h * n2, n2)
                if slot == 1:
                    out_ref[:, cols] = (
                        (out_ref[:, cols] + contrib) * (sx_ref[0] * sw_ref[0])
                    )
                else:
                    out_ref[:, cols] = out_ref[:, cols] + contrib

        for s in sends:
            s.wait_send()

    return pl.pallas_call(
        body,
        out_shape=jax.ShapeDtypeStruct((m_per, n), jnp.float32),
        in_specs=[
            pl.BlockSpec(memory_space=pltpu.VMEM),
            pl.BlockSpec(memory_space=pltpu.VMEM),
            pl.BlockSpec(memory_space=pltpu.SMEM),
            pl.BlockSpec(memory_space=pltpu.SMEM),
        ],
        out_specs=pl.BlockSpec(memory_space=pltpu.VMEM),
        scratch_shapes=[
            pltpu.VMEM((N_DEV - 1, m_per, k_per), F8X),
            pltpu.VMEM((k_per, n), F8W),
            pltpu.VMEM((N_DEV - 1, m_per, k_per), F8X),
            pltpu.VMEM((N_DEV - 1, k_per, n), F8W),
            pltpu.SemaphoreType.DMA((N_DEV - 1,)),
            pltpu.SemaphoreType.DMA((N_DEV - 1, 2)),
            pltpu.SemaphoreType.DMA((N_DEV - 1,)),
            pltpu.SemaphoreType.DMA((N_DEV - 1, 2)),
        ],
        compiler_params=pltpu.CompilerParams(
            collective_id=0,
            vmem_limit_bytes=62 * 1024 * 1024,
        ),
    )(x, w_mat, scale_x, scale_w)
